# Initial kernel scaffold; baseline (speedup 1.0000x reference)
import jax
import jax.numpy as jnp
from jax.experimental import pallas as pl


def kernel(node_states, from_idx, to_idx, edge_features, Wm1, bm1, Wm2, bm2, Wr1, br1, Wr2, br2, Wn, bn):
    raise NotImplementedError("write your pallas kernel here")


# speedup vs baseline: 2.1914x; 2.1914x over previous
import functools

import jax
import jax.numpy as jnp
from jax import lax
from jax.experimental import pallas as pl
from jax.experimental.pallas import tpu as pltpu
from jax.experimental.pallas import tpu_sc as plsc

N = 10000
E = 320000
D = 128
DE = 16
H = 256
M = 128

NC = 2
NS = 16
NW = NC * NS


_R_PER_W = (2 * E) // NW
_CH = 80
_NG = _R_PER_W // _CH
_NP = _NG // 2


def _gather_body(table_hbm, idx_hbm, out_hbm, idx_res, rows0, rows1, s0, s1):
    c = lax.axis_index("c")
    s = lax.axis_index("s")
    wid = s * NC + c
    base0 = wid * _R_PER_W

    pltpu.sync_copy(idx_hbm.at[wid], idx_res)

    pltpu.async_copy(table_hbm.at[idx_res.at[0]], rows0, s0)

    def step(p, _):
        g = 2 * p
        pltpu.make_async_copy(table_hbm.at[idx_res.at[g]], rows0, s0).wait()
        pltpu.async_copy(table_hbm.at[idx_res.at[g + 1]], rows1, s1)
        pltpu.sync_copy(rows0, out_hbm.at[pl.ds(base0 + g * _CH, _CH)])
        pltpu.make_async_copy(table_hbm.at[idx_res.at[g + 1]], rows1, s1).wait()

        @pl.when(g + 2 < _NG)
        def _():
            pltpu.async_copy(table_hbm.at[idx_res.at[g + 2]], rows0, s0)

        pltpu.sync_copy(rows1, out_hbm.at[pl.ds(base0 + (g + 1) * _CH, _CH)])
        return 0

    lax.fori_loop(0, _NP, step, 0)


def _sc_gather(table, idx):
    mesh = plsc.VectorSubcoreMesh(core_axis_name="c", subcore_axis_name="s")
    f = functools.partial(
        pl.kernel,
        mesh=mesh,
        out_type=jax.ShapeDtypeStruct((2 * E, D), jnp.float32),
        scratch_types=[
            pltpu.VMEM((_NG, _CH), jnp.int32),
            pltpu.VMEM((_CH, D), jnp.float32),
            pltpu.VMEM((_CH, D), jnp.float32),
            pltpu.SemaphoreType.DMA,
            pltpu.SemaphoreType.DMA,
        ],
    )(_gather_body)
    return f(table, idx)


_NPAD = 10240
_N_PER_S = _NPAD // NS
_IC = 8
_NK = 32


def _scatter_body(msg_hbm, sidx_hbm, z_hbm, out_hbm,
                  idx_v, msg0, msg1, s0, s1, acc_sh):
    c = lax.axis_index("c")
    s = lax.axis_index("s")
    wid = s * NC + c
    base0 = wid * _R_PER_W

    @pl.when(s == 0)
    def _():
        pltpu.sync_copy(z_hbm, acc_sh)

    plsc.subcore_barrier()

    pltpu.async_copy(msg_hbm.at[pl.ds(base0, _CH)], msg0, s0)
    msgs = (msg0, msg1)
    sems = (s0, s1)

    def step(k, _):
        pltpu.sync_copy(sidx_hbm.at[wid, k], idx_v)
        for j in range(_IC):
            g = k * _IC + j
            b = j % 2
            pltpu.make_async_copy(
                msg_hbm.at[pl.ds(base0 + g * _CH, _CH)], msgs[b], sems[b]).wait()
            pltpu.async_copy(
                msg_hbm.at[pl.ds(base0 + (g + 1) * _CH, _CH)],
                msgs[1 - b], sems[1 - b])
            pltpu.sync_copy(msgs[b], acc_sh.at[idx_v.at[j]], add=True)
        return 0

    lax.fori_loop(0, _NK - 1, step, 0)

    pltpu.sync_copy(sidx_hbm.at[wid, _NK - 1], idx_v)
    g0 = (_NK - 1) * _IC
    pltpu.make_async_copy(
        msg_hbm.at[pl.ds(base0 + g0 * _CH, _CH)], msg0, s0).wait()
    pltpu.async_copy(
        msg_hbm.at[pl.ds(base0 + (g0 + 1) * _CH, _CH)], msg1, s1)
    pltpu.sync_copy(msg0, acc_sh.at[idx_v.at[0]], add=True)
    pltpu.make_async_copy(
        msg_hbm.at[pl.ds(base0 + (g0 + 1) * _CH, _CH)], msg1, s1).wait()
    pltpu.sync_copy(msg1, acc_sh.at[idx_v.at[1]], add=True)

    plsc.subcore_barrier()
    pltpu.sync_copy(
        acc_sh.at[pl.ds(s * _N_PER_S, _N_PER_S)],
        out_hbm.at[pl.ds(c * _NPAD + s * _N_PER_S, _N_PER_S)],
    )


def _sc_scatter(msg, sidx, zeros_nm):
    mesh = plsc.VectorSubcoreMesh(core_axis_name="c", subcore_axis_name="s")
    f = functools.partial(
        pl.kernel,
        mesh=mesh,
        out_type=jax.ShapeDtypeStruct((NC * _NPAD, M), jnp.float32),
        scratch_types=[
            pltpu.VMEM((_IC, _CH), jnp.int32),
            pltpu.VMEM((_CH, M), jnp.float32),
            pltpu.VMEM((_CH, M), jnp.float32),
            pltpu.SemaphoreType.DMA,
            pltpu.SemaphoreType.DMA,
            pltpu.VMEM_SHARED((_NPAD, M), jnp.float32),
        ],
    )(_scatter_body)
    return f(msg, sidx, zeros_nm)


_BE = 512


def _msg_body(ft, ef, wm1f, wm1t, wm1e, bm1, wm2, bm2,
              wr1f, wr1t, wr1e, br1, wr2, br2, out):
    f = ft[0]
    t = ft[1]
    e = ef[...]
    hf = jnp.maximum(
        jnp.dot(f, wm1f[...], preferred_element_type=jnp.float32)
        + jnp.dot(t, wm1t[...], preferred_element_type=jnp.float32)
        + jnp.dot(e, wm1e[...], preferred_element_type=jnp.float32)
        + bm1[...], 0.0)
    out[0] = jnp.dot(hf, wm2[...], preferred_element_type=jnp.float32) + bm2[...]
    hr = jnp.maximum(
        jnp.dot(t, wr1f[...], preferred_element_type=jnp.float32)
        + jnp.dot(f, wr1t[...], preferred_element_type=jnp.float32)
        + jnp.dot(e, wr1e[...], preferred_element_type=jnp.float32)
        + br1[...], 0.0)
    out[1] = jnp.dot(hr, wr2[...], preferred_element_type=jnp.float32) + br2[...]


def _tc_messages(ft, ef, Wm1, bm1, Wm2, bm2, Wr1, br1, Wr2, br2):
    grid = (E // _BE,)
    full = lambda a: pl.BlockSpec(a.shape, lambda i: (0,) * a.ndim)
    weights = [Wm1[:D], Wm1[D:2 * D], Wm1[2 * D:], bm1.reshape(1, H),
               Wm2, bm2.reshape(1, M),
               Wr1[:D], Wr1[D:2 * D], Wr1[2 * D:], br1.reshape(1, H),
               Wr2, br2.reshape(1, M)]
    return pl.pallas_call(
        _msg_body,
        grid=grid,
        in_specs=[pl.BlockSpec((2, _BE, D), lambda i: (0, i, 0)),
                  pl.BlockSpec((_BE, DE), lambda i: (i, 0))]
                 + [full(w) for w in weights],
        out_specs=pl.BlockSpec((2, _BE, M), lambda i: (0, i, 0)),
        out_shape=jax.ShapeDtypeStruct((2, E, M), jnp.float32),
    )(ft, ef, *weights)


_BN = 1000


def _update_body(ns, p, wa, wb, bn, out):
    x = ns[...]
    agg = p[0] + p[1]
    out[...] = (x + bn[...]
                + jnp.dot(x, wa[...], preferred_element_type=jnp.float32)
                + jnp.dot(agg, wb[...], preferred_element_type=jnp.float32))


def _tc_update(ns, parts, Wn, bn):
    grid = (N // _BN,)
    full = lambda a: pl.BlockSpec(a.shape, lambda i: (0,) * a.ndim)
    wa, wb, bnr = Wn[:D], Wn[D:], bn.reshape(1, D)
    return pl.pallas_call(
        _update_body,
        grid=grid,
        in_specs=[pl.BlockSpec((_BN, D), lambda i: (i, 0)),
                  pl.BlockSpec((2, _BN, M), lambda i: (0, i, 0)),
                  full(wa), full(wb), full(bnr)],
        out_specs=pl.BlockSpec((_BN, D), lambda i: (i, 0)),
        out_shape=jax.ShapeDtypeStruct((N, D), jnp.float32),
    )(ns, parts, wa, wb, bnr)


def kernel(node_states, from_idx, to_idx, edge_features,
           Wm1, bm1, Wm2, bm2, Wr1, br1, Wr2, br2, Wn, bn):
    gidx = jnp.concatenate([from_idx, to_idx])
    sidx = jnp.concatenate([to_idx, from_idx])
    ft = _sc_gather(node_states, gidx.reshape(NW, _NG, _CH)).reshape(2, E, D)
    msg = _tc_messages(ft, edge_features, Wm1, bm1, Wm2, bm2, Wr1, br1, Wr2, br2)
    sidx4 = jnp.pad(sidx.reshape(NW, _NG, _CH),
                    ((0, 0), (0, _NK * _IC - _NG), (0, 0))
                    ).reshape(NW, _NK, _IC, _CH)
    parts = _sc_scatter(msg.reshape(2 * E, M), sidx4,
                        jnp.zeros((_NPAD, M), jnp.float32))
    parts = parts.reshape(NC, _NPAD, M)
    return _tc_update(node_states, parts, Wn, bn)


# speedup vs baseline: 2.2238x; 1.0147x over previous
import functools

import jax
import jax.numpy as jnp
from jax import lax
from jax.experimental import pallas as pl
from jax.experimental.pallas import tpu as pltpu
from jax.experimental.pallas import tpu_sc as plsc

N = 10000
E = 320000
D = 128
DE = 16
H = 256
M = 128

NC = 2
NS = 16
NW = NC * NS


_R_PER_W = (2 * E) // NW
_CH = 80
_NG = _R_PER_W // _CH
_NP = _NG // 2


def _gather_body(table_hbm, idx_hbm, out_hbm, idx_res, rows0, rows1, s0, s1):
    c = lax.axis_index("c")
    s = lax.axis_index("s")
    wid = s * NC + c
    base0 = wid * _R_PER_W

    pltpu.sync_copy(idx_hbm.at[wid], idx_res)

    pltpu.async_copy(table_hbm.at[idx_res.at[0]], rows0, s0)

    def step(p, _):
        g = 2 * p
        pltpu.make_async_copy(table_hbm.at[idx_res.at[g]], rows0, s0).wait()
        pltpu.async_copy(table_hbm.at[idx_res.at[g + 1]], rows1, s1)
        pltpu.sync_copy(rows0, out_hbm.at[pl.ds(base0 + g * _CH, _CH)])
        pltpu.make_async_copy(table_hbm.at[idx_res.at[g + 1]], rows1, s1).wait()

        @pl.when(g + 2 < _NG)
        def _():
            pltpu.async_copy(table_hbm.at[idx_res.at[g + 2]], rows0, s0)

        pltpu.sync_copy(rows1, out_hbm.at[pl.ds(base0 + (g + 1) * _CH, _CH)])
        return 0

    lax.fori_loop(0, _NP, step, 0)


def _sc_gather(table, idx):
    mesh = plsc.VectorSubcoreMesh(core_axis_name="c", subcore_axis_name="s")
    f = functools.partial(
        pl.kernel,
        mesh=mesh,
        out_type=jax.ShapeDtypeStruct((2 * E, D), jnp.float32),
        scratch_types=[
            pltpu.VMEM((_NG, _CH), jnp.int32),
            pltpu.VMEM((_CH, D), jnp.float32),
            pltpu.VMEM((_CH, D), jnp.float32),
            pltpu.SemaphoreType.DMA,
            pltpu.SemaphoreType.DMA,
        ],
    )(_gather_body)
    return f(table, idx)


_NPAD = 10240
_N_PER_S = _NPAD // NS
_IC = 8
_NK = 32


def _scatter_body(msg_hbm, sidx_hbm, z_hbm, out_hbm,
                  idx_v, msg0, msg1, s0, s1, acc_sh):
    c = lax.axis_index("c")
    s = lax.axis_index("s")
    wid = s * NC + c
    base0 = wid * _R_PER_W

    @pl.when(s == 0)
    def _():
        pltpu.sync_copy(z_hbm, acc_sh)

    plsc.subcore_barrier()

    pltpu.async_copy(msg_hbm.at[pl.ds(base0, _CH)], msg0, s0)
    msgs = (msg0, msg1)
    sems = (s0, s1)

    def step(k, _):
        pltpu.sync_copy(sidx_hbm.at[wid, k], idx_v)
        for j in range(_IC):
            g = k * _IC + j
            b = j % 2
            pltpu.make_async_copy(
                msg_hbm.at[pl.ds(base0 + g * _CH, _CH)], msgs[b], sems[b]).wait()
            pltpu.async_copy(
                msg_hbm.at[pl.ds(base0 + (g + 1) * _CH, _CH)],
                msgs[1 - b], sems[1 - b])
            pltpu.sync_copy(msgs[b], acc_sh.at[idx_v.at[j]], add=True)
        return 0

    lax.fori_loop(0, _NK - 1, step, 0)

    pltpu.sync_copy(sidx_hbm.at[wid, _NK - 1], idx_v)
    g0 = (_NK - 1) * _IC
    pltpu.make_async_copy(
        msg_hbm.at[pl.ds(base0 + g0 * _CH, _CH)], msg0, s0).wait()
    pltpu.async_copy(
        msg_hbm.at[pl.ds(base0 + (g0 + 1) * _CH, _CH)], msg1, s1)
    pltpu.sync_copy(msg0, acc_sh.at[idx_v.at[0]], add=True)
    pltpu.make_async_copy(
        msg_hbm.at[pl.ds(base0 + (g0 + 1) * _CH, _CH)], msg1, s1).wait()
    pltpu.sync_copy(msg1, acc_sh.at[idx_v.at[1]], add=True)

    plsc.subcore_barrier()
    pltpu.sync_copy(
        acc_sh.at[pl.ds(s * _N_PER_S, _N_PER_S)],
        out_hbm.at[pl.ds(c * _NPAD + s * _N_PER_S, _N_PER_S)],
    )


def _sc_scatter(msg, sidx, zeros_nm):
    mesh = plsc.VectorSubcoreMesh(core_axis_name="c", subcore_axis_name="s")
    f = functools.partial(
        pl.kernel,
        mesh=mesh,
        out_type=jax.ShapeDtypeStruct((NC * _NPAD, M), jnp.float32),
        scratch_types=[
            pltpu.VMEM((_IC, _CH), jnp.int32),
            pltpu.VMEM((_CH, M), jnp.float32),
            pltpu.VMEM((_CH, M), jnp.float32),
            pltpu.SemaphoreType.DMA,
            pltpu.SemaphoreType.DMA,
            pltpu.VMEM_SHARED((_NPAD, M), jnp.float32),
        ],
    )(_scatter_body)
    return f(msg, sidx, zeros_nm)


_BE = 512


def _msg_body(ft, ef, wm1f, wm1t, wm1e, bm1, wm2, bm2,
              wr1f, wr1t, wr1e, br1, wr2, br2, out):
    f = ft[0].astype(jnp.bfloat16)
    t = ft[1].astype(jnp.bfloat16)
    e = ef[...]
    hf = jnp.maximum(
        jnp.dot(f, wm1f[...], preferred_element_type=jnp.float32)
        + jnp.dot(t, wm1t[...], preferred_element_type=jnp.float32)
        + jnp.dot(e, wm1e[...], preferred_element_type=jnp.float32)
        + bm1[...], 0.0).astype(jnp.bfloat16)
    out[0] = jnp.dot(hf, wm2[...], preferred_element_type=jnp.float32) + bm2[...]
    hr = jnp.maximum(
        jnp.dot(t, wr1f[...], preferred_element_type=jnp.float32)
        + jnp.dot(f, wr1t[...], preferred_element_type=jnp.float32)
        + jnp.dot(e, wr1e[...], preferred_element_type=jnp.float32)
        + br1[...], 0.0).astype(jnp.bfloat16)
    out[1] = jnp.dot(hr, wr2[...], preferred_element_type=jnp.float32) + br2[...]


def _tc_messages(ft, ef, Wm1, bm1, Wm2, bm2, Wr1, br1, Wr2, br2):
    grid = (E // _BE,)
    full = lambda a: pl.BlockSpec(a.shape, lambda i: (0,) * a.ndim)
    b16 = lambda a: a.astype(jnp.bfloat16)
    weights = [b16(Wm1[:D]), b16(Wm1[D:2 * D]), b16(Wm1[2 * D:]),
               bm1.reshape(1, H), b16(Wm2), bm2.reshape(1, M),
               b16(Wr1[:D]), b16(Wr1[D:2 * D]), b16(Wr1[2 * D:]),
               br1.reshape(1, H), b16(Wr2), br2.reshape(1, M)]
    return pl.pallas_call(
        _msg_body,
        grid=grid,
        in_specs=[pl.BlockSpec((2, _BE, D), lambda i: (0, i, 0)),
                  pl.BlockSpec((_BE, DE), lambda i: (i, 0))]
                 + [full(w) for w in weights],
        out_specs=pl.BlockSpec((2, _BE, M), lambda i: (0, i, 0)),
        out_shape=jax.ShapeDtypeStruct((2, E, M), jnp.float32),
    )(ft, ef, *weights)


_BN = 1000


def _update_body(ns, p, wa, wb, bn, out):
    x = ns[...]
    agg = p[0] + p[1]
    out[...] = (x + bn[...]
                + jnp.dot(x, wa[...], preferred_element_type=jnp.float32)
                + jnp.dot(agg, wb[...], preferred_element_type=jnp.float32))


def _tc_update(ns, parts, Wn, bn):
    grid = (N // _BN,)
    full = lambda a: pl.BlockSpec(a.shape, lambda i: (0,) * a.ndim)
    wa, wb, bnr = Wn[:D], Wn[D:], bn.reshape(1, D)
    return pl.pallas_call(
        _update_body,
        grid=grid,
        in_specs=[pl.BlockSpec((_BN, D), lambda i: (i, 0)),
                  pl.BlockSpec((2, _BN, M), lambda i: (0, i, 0)),
                  full(wa), full(wb), full(bnr)],
        out_specs=pl.BlockSpec((_BN, D), lambda i: (i, 0)),
        out_shape=jax.ShapeDtypeStruct((N, D), jnp.float32),
    )(ns, parts, wa, wb, bnr)


def kernel(node_states, from_idx, to_idx, edge_features,
           Wm1, bm1, Wm2, bm2, Wr1, br1, Wr2, br2, Wn, bn):
    gidx = jnp.concatenate([from_idx, to_idx])
    sidx = jnp.concatenate([to_idx, from_idx])
    ft = _sc_gather(node_states, gidx.reshape(NW, _NG, _CH)).reshape(2, E, D)
    msg = _tc_messages(ft, edge_features.astype(jnp.bfloat16),
                       Wm1, bm1, Wm2, bm2, Wr1, br1, Wr2, br2)
    sidx4 = jnp.pad(sidx.reshape(NW, _NG, _CH),
                    ((0, 0), (0, _NK * _IC - _NG), (0, 0))
                    ).reshape(NW, _NK, _IC, _CH)
    parts = _sc_scatter(msg.reshape(2 * E, M), sidx4,
                        jnp.zeros((_NPAD, M), jnp.float32))
    parts = parts.reshape(NC, _NPAD, M)
    return _tc_update(node_states, parts, Wn, bn)


# speedup vs baseline: 2.6216x; 1.1789x over previous
import functools

import jax
import jax.numpy as jnp
from jax import lax
from jax.experimental import pallas as pl
from jax.experimental.pallas import tpu as pltpu
from jax.experimental.pallas import tpu_sc as plsc

N = 10000
E = 320000
D = 128
DE = 16
H = 256
M = 128

NC = 2
NS = 16
NW = NC * NS


_R_PER_W = (2 * E) // NW
_CH = 80
_NG = _R_PER_W // _CH
_NP = _NG // 2


def _gather_body(table_hbm, idx_hbm, out_hbm, idx_res, rows0, rows1, s0, s1):
    c = lax.axis_index("c")
    s = lax.axis_index("s")
    wid = s * NC + c
    base0 = wid * _R_PER_W

    pltpu.sync_copy(idx_hbm.at[wid], idx_res)

    pltpu.async_copy(table_hbm.at[idx_res.at[0]], rows0, s0)

    def step(p, _):
        g = 2 * p
        pltpu.make_async_copy(table_hbm.at[idx_res.at[g]], rows0, s0).wait()
        pltpu.async_copy(table_hbm.at[idx_res.at[g + 1]], rows1, s1)
        pltpu.sync_copy(rows0, out_hbm.at[pl.ds(base0 + g * _CH, _CH)])
        pltpu.make_async_copy(table_hbm.at[idx_res.at[g + 1]], rows1, s1).wait()

        @pl.when(g + 2 < _NG)
        def _():
            pltpu.async_copy(table_hbm.at[idx_res.at[g + 2]], rows0, s0)

        pltpu.sync_copy(rows1, out_hbm.at[pl.ds(base0 + (g + 1) * _CH, _CH)])
        return 0

    lax.fori_loop(0, _NP, step, 0)


def _sc_gather(table, idx):
    mesh = plsc.VectorSubcoreMesh(core_axis_name="c", subcore_axis_name="s")
    f = functools.partial(
        pl.kernel,
        mesh=mesh,
        out_type=jax.ShapeDtypeStruct((2 * E, D), jnp.float32),
        scratch_types=[
            pltpu.VMEM((_NG, _CH), jnp.int32),
            pltpu.VMEM((_CH, D), jnp.float32),
            pltpu.VMEM((_CH, D), jnp.float32),
            pltpu.SemaphoreType.DMA,
            pltpu.SemaphoreType.DMA,
        ],
    )(_gather_body)
    return f(table, idx)


_NPAD = 10240
_N_PER_S = _NPAD // NS
_IC = 8
_NK = 32


def _scatter_body(msg_hbm, sidx_hbm, z_hbm, out_hbm,
                  idx_v, msg0, msg1, s0, s1, acc_sh):
    c = lax.axis_index("c")
    s = lax.axis_index("s")
    wid = s * NC + c
    base0 = wid * _R_PER_W

    @pl.when(s == 0)
    def _():
        pltpu.sync_copy(z_hbm, acc_sh)

    plsc.subcore_barrier()

    pltpu.async_copy(msg_hbm.at[pl.ds(base0, _CH)], msg0, s0)
    msgs = (msg0, msg1)
    sems = (s0, s1)

    def step(k, _):
        pltpu.sync_copy(sidx_hbm.at[wid, k], idx_v)
        for j in range(_IC):
            g = k * _IC + j
            b = j % 2
            pltpu.make_async_copy(
                msg_hbm.at[pl.ds(base0 + g * _CH, _CH)], msgs[b], sems[b]).wait()
            pltpu.async_copy(
                msg_hbm.at[pl.ds(base0 + (g + 1) * _CH, _CH)],
                msgs[1 - b], sems[1 - b])
            pltpu.sync_copy(msgs[b], acc_sh.at[idx_v.at[j]], add=True)
        return 0

    lax.fori_loop(0, _NK - 1, step, 0)

    pltpu.sync_copy(sidx_hbm.at[wid, _NK - 1], idx_v)
    g0 = (_NK - 1) * _IC
    pltpu.make_async_copy(
        msg_hbm.at[pl.ds(base0 + g0 * _CH, _CH)], msg0, s0).wait()
    pltpu.async_copy(
        msg_hbm.at[pl.ds(base0 + (g0 + 1) * _CH, _CH)], msg1, s1)
    pltpu.sync_copy(msg0, acc_sh.at[idx_v.at[0]], add=True)
    pltpu.make_async_copy(
        msg_hbm.at[pl.ds(base0 + (g0 + 1) * _CH, _CH)], msg1, s1).wait()
    pltpu.sync_copy(msg1, acc_sh.at[idx_v.at[1]], add=True)

    plsc.subcore_barrier()
    pltpu.sync_copy(
        acc_sh.at[pl.ds(s * _N_PER_S, _N_PER_S)],
        out_hbm.at[pl.ds(c * _NPAD + s * _N_PER_S, _N_PER_S)],
    )


def _sc_scatter(msg, sidx, zeros_nm):
    mesh = plsc.VectorSubcoreMesh(core_axis_name="c", subcore_axis_name="s")
    f = functools.partial(
        pl.kernel,
        mesh=mesh,
        out_type=jax.ShapeDtypeStruct((NC * _NPAD, M), jnp.float32),
        scratch_types=[
            pltpu.VMEM((_IC, _CH), jnp.int32),
            pltpu.VMEM((_CH, M), jnp.float32),
            pltpu.VMEM((_CH, M), jnp.float32),
            pltpu.SemaphoreType.DMA,
            pltpu.SemaphoreType.DMA,
            pltpu.VMEM_SHARED((_NPAD, M), jnp.float32),
        ],
    )(_scatter_body)
    return f(msg, sidx, zeros_nm)


_BE = 2000


def _msg_body(ft, ef, wm1f, wm1t, wm1e, bm1, wm2, bm2,
              wr1f, wr1t, wr1e, br1, wr2, br2, out):
    f = ft[0].astype(jnp.bfloat16)
    t = ft[1].astype(jnp.bfloat16)
    e = ef[...]
    hf = jnp.maximum(
        jnp.dot(f, wm1f[...], preferred_element_type=jnp.float32)
        + jnp.dot(t, wm1t[...], preferred_element_type=jnp.float32)
        + jnp.dot(e, wm1e[...], preferred_element_type=jnp.float32)
        + bm1[...], 0.0).astype(jnp.bfloat16)
    out[0] = jnp.dot(hf, wm2[...], preferred_element_type=jnp.float32) + bm2[...]
    hr = jnp.maximum(
        jnp.dot(t, wr1f[...], preferred_element_type=jnp.float32)
        + jnp.dot(f, wr1t[...], preferred_element_type=jnp.float32)
        + jnp.dot(e, wr1e[...], preferred_element_type=jnp.float32)
        + br1[...], 0.0).astype(jnp.bfloat16)
    out[1] = jnp.dot(hr, wr2[...], preferred_element_type=jnp.float32) + br2[...]


def _tc_messages(ft, ef, Wm1, bm1, Wm2, bm2, Wr1, br1, Wr2, br2):
    grid = (E // _BE,)
    full = lambda a: pl.BlockSpec(a.shape, lambda i: (0,) * a.ndim)
    b16 = lambda a: a.astype(jnp.bfloat16)
    weights = [b16(Wm1[:D]), b16(Wm1[D:2 * D]), b16(Wm1[2 * D:]),
               bm1.reshape(1, H), b16(Wm2), bm2.reshape(1, M),
               b16(Wr1[:D]), b16(Wr1[D:2 * D]), b16(Wr1[2 * D:]),
               br1.reshape(1, H), b16(Wr2), br2.reshape(1, M)]
    return pl.pallas_call(
        _msg_body,
        grid=grid,
        in_specs=[pl.BlockSpec((2, _BE, D), lambda i: (0, i, 0)),
                  pl.BlockSpec((_BE, DE), lambda i: (i, 0))]
                 + [full(w) for w in weights],
        out_specs=pl.BlockSpec((2, _BE, M), lambda i: (0, i, 0)),
        out_shape=jax.ShapeDtypeStruct((2, E, M), jnp.float32),
    )(ft, ef, *weights)


_BN = 1000


def _update_body(ns, p, wa, wb, bn, out):
    x = ns[...]
    agg = p[0] + p[1]
    out[...] = (x + bn[...]
                + jnp.dot(x, wa[...], preferred_element_type=jnp.float32)
                + jnp.dot(agg, wb[...], preferred_element_type=jnp.float32))


def _tc_update(ns, parts, Wn, bn):
    grid = (N // _BN,)
    full = lambda a: pl.BlockSpec(a.shape, lambda i: (0,) * a.ndim)
    wa, wb, bnr = Wn[:D], Wn[D:], bn.reshape(1, D)
    return pl.pallas_call(
        _update_body,
        grid=grid,
        in_specs=[pl.BlockSpec((_BN, D), lambda i: (i, 0)),
                  pl.BlockSpec((2, _BN, M), lambda i: (0, i, 0)),
                  full(wa), full(wb), full(bnr)],
        out_specs=pl.BlockSpec((_BN, D), lambda i: (i, 0)),
        out_shape=jax.ShapeDtypeStruct((N, D), jnp.float32),
    )(ns, parts, wa, wb, bnr)


def kernel(node_states, from_idx, to_idx, edge_features,
           Wm1, bm1, Wm2, bm2, Wr1, br1, Wr2, br2, Wn, bn):
    gidx = jnp.concatenate([from_idx, to_idx])
    sidx = jnp.concatenate([to_idx, from_idx])
    ft = _sc_gather(node_states, gidx.reshape(NW, _NG, _CH)).reshape(2, E, D)
    msg = _tc_messages(ft, edge_features.astype(jnp.bfloat16),
                       Wm1, bm1, Wm2, bm2, Wr1, br1, Wr2, br2)
    sidx4 = jnp.pad(sidx.reshape(NW, _NG, _CH),
                    ((0, 0), (0, _NK * _IC - _NG), (0, 0))
                    ).reshape(NW, _NK, _IC, _CH)
    parts = _sc_scatter(msg.reshape(2 * E, M), sidx4,
                        jnp.zeros((_NPAD, M), jnp.float32))
    parts = parts.reshape(NC, _NPAD, M)
    return _tc_update(node_states, parts, Wn, bn)


# speedup vs baseline: 2.6992x; 1.0296x over previous
import functools

import jax
import jax.numpy as jnp
from jax import lax
from jax.experimental import pallas as pl
from jax.experimental.pallas import tpu as pltpu
from jax.experimental.pallas import tpu_sc as plsc

N = 10000
E = 320000
D = 128
DE = 16
H = 256
M = 128

NC = 2
NS = 16
NW = NC * NS


_R_PER_W = (2 * E) // NW
_CH = 80
_NG = _R_PER_W // _CH
_NP = _NG // 2


def _gather_body(table_hbm, idx_hbm, out_hbm, idx_res, rows0, rows1, s0, s1):
    c = lax.axis_index("c")
    s = lax.axis_index("s")
    wid = s * NC + c
    base0 = wid * _R_PER_W

    pltpu.sync_copy(idx_hbm.at[wid], idx_res)

    pltpu.async_copy(table_hbm.at[idx_res.at[0]], rows0, s0)

    def step(p, _):
        g = 2 * p
        pltpu.make_async_copy(table_hbm.at[idx_res.at[g]], rows0, s0).wait()
        pltpu.async_copy(table_hbm.at[idx_res.at[g + 1]], rows1, s1)
        pltpu.sync_copy(rows0, out_hbm.at[pl.ds(base0 + g * _CH, _CH)])
        pltpu.make_async_copy(table_hbm.at[idx_res.at[g + 1]], rows1, s1).wait()

        @pl.when(g + 2 < _NG)
        def _():
            pltpu.async_copy(table_hbm.at[idx_res.at[g + 2]], rows0, s0)

        pltpu.sync_copy(rows1, out_hbm.at[pl.ds(base0 + (g + 1) * _CH, _CH)])
        return 0

    lax.fori_loop(0, _NP, step, 0)


def _sc_gather(table, idx):
    mesh = plsc.VectorSubcoreMesh(core_axis_name="c", subcore_axis_name="s")
    f = functools.partial(
        pl.kernel,
        mesh=mesh,
        out_type=jax.ShapeDtypeStruct((2 * E, D), jnp.float32),
        scratch_types=[
            pltpu.VMEM((_NG, _CH), jnp.int32),
            pltpu.VMEM((_CH, D), jnp.float32),
            pltpu.VMEM((_CH, D), jnp.float32),
            pltpu.SemaphoreType.DMA,
            pltpu.SemaphoreType.DMA,
        ],
    )(_gather_body)
    return f(table, idx)


_NPAD = 10240
_N_PER_S = _NPAD // NS
_IC = 8
_NK = 32


def _scatter_body(msg_hbm, sidx_hbm, z_hbm, out_hbm,
                  idx_v, msg0, msg1, s0, s1, acc_sh):
    c = lax.axis_index("c")
    s = lax.axis_index("s")
    wid = s * NC + c
    base0 = wid * _R_PER_W

    @pl.when(s == 0)
    def _():
        pltpu.sync_copy(z_hbm, acc_sh)

    plsc.subcore_barrier()

    pltpu.async_copy(msg_hbm.at[pl.ds(base0, _CH)], msg0, s0)
    msgs = (msg0, msg1)
    sems = (s0, s1)

    def step(k, _):
        pltpu.sync_copy(sidx_hbm.at[wid, k], idx_v)
        for j in range(_IC):
            g = k * _IC + j
            b = j % 2
            pltpu.make_async_copy(
                msg_hbm.at[pl.ds(base0 + g * _CH, _CH)], msgs[b], sems[b]).wait()
            pltpu.async_copy(
                msg_hbm.at[pl.ds(base0 + (g + 1) * _CH, _CH)],
                msgs[1 - b], sems[1 - b])
            pltpu.sync_copy(msgs[b], acc_sh.at[idx_v.at[j]], add=True)
        return 0

    lax.fori_loop(0, _NK - 1, step, 0)

    pltpu.sync_copy(sidx_hbm.at[wid, _NK - 1], idx_v)
    g0 = (_NK - 1) * _IC
    pltpu.make_async_copy(
        msg_hbm.at[pl.ds(base0 + g0 * _CH, _CH)], msg0, s0).wait()
    pltpu.async_copy(
        msg_hbm.at[pl.ds(base0 + (g0 + 1) * _CH, _CH)], msg1, s1)
    pltpu.sync_copy(msg0, acc_sh.at[idx_v.at[0]], add=True)
    pltpu.make_async_copy(
        msg_hbm.at[pl.ds(base0 + (g0 + 1) * _CH, _CH)], msg1, s1).wait()
    pltpu.sync_copy(msg1, acc_sh.at[idx_v.at[1]], add=True)

    plsc.subcore_barrier()
    pltpu.sync_copy(
        acc_sh.at[pl.ds(s * _N_PER_S, _N_PER_S)],
        out_hbm.at[pl.ds(c * _NPAD + s * _N_PER_S, _N_PER_S)],
    )


def _sc_scatter(msg, sidx, zeros_nm):
    mesh = plsc.VectorSubcoreMesh(core_axis_name="c", subcore_axis_name="s")
    f = functools.partial(
        pl.kernel,
        mesh=mesh,
        out_type=jax.ShapeDtypeStruct((NC * _NPAD, M), jnp.float32),
        scratch_types=[
            pltpu.VMEM((_IC, _CH), jnp.int32),
            pltpu.VMEM((_CH, M), jnp.float32),
            pltpu.VMEM((_CH, M), jnp.float32),
            pltpu.SemaphoreType.DMA,
            pltpu.SemaphoreType.DMA,
            pltpu.VMEM_SHARED((_NPAD, M), jnp.float32),
        ],
    )(_scatter_body)
    return f(msg, sidx, zeros_nm)


_BE = 4000


def _msg_body(ft, ef, wm1f, wm1t, wm1e, bm1, wm2, bm2,
              wr1f, wr1t, wr1e, br1, wr2, br2, out):
    f = ft[0].astype(jnp.bfloat16)
    t = ft[1].astype(jnp.bfloat16)
    e = ef[...]
    hf = jnp.maximum(
        jnp.dot(f, wm1f[...], preferred_element_type=jnp.float32)
        + jnp.dot(t, wm1t[...], preferred_element_type=jnp.float32)
        + jnp.dot(e, wm1e[...], preferred_element_type=jnp.float32)
        + bm1[...], 0.0).astype(jnp.bfloat16)
    out[0] = jnp.dot(hf, wm2[...], preferred_element_type=jnp.float32) + bm2[...]
    hr = jnp.maximum(
        jnp.dot(t, wr1f[...], preferred_element_type=jnp.float32)
        + jnp.dot(f, wr1t[...], preferred_element_type=jnp.float32)
        + jnp.dot(e, wr1e[...], preferred_element_type=jnp.float32)
        + br1[...], 0.0).astype(jnp.bfloat16)
    out[1] = jnp.dot(hr, wr2[...], preferred_element_type=jnp.float32) + br2[...]


def _tc_messages(ft, ef, Wm1, bm1, Wm2, bm2, Wr1, br1, Wr2, br2):
    grid = (E // _BE,)
    full = lambda a: pl.BlockSpec(a.shape, lambda i: (0,) * a.ndim)
    b16 = lambda a: a.astype(jnp.bfloat16)
    weights = [b16(Wm1[:D]), b16(Wm1[D:2 * D]), b16(Wm1[2 * D:]),
               bm1.reshape(1, H), b16(Wm2), bm2.reshape(1, M),
               b16(Wr1[:D]), b16(Wr1[D:2 * D]), b16(Wr1[2 * D:]),
               br1.reshape(1, H), b16(Wr2), br2.reshape(1, M)]
    return pl.pallas_call(
        _msg_body,
        grid=grid,
        in_specs=[pl.BlockSpec((2, _BE, D), lambda i: (0, i, 0)),
                  pl.BlockSpec((_BE, DE), lambda i: (i, 0))]
                 + [full(w) for w in weights],
        out_specs=pl.BlockSpec((2, _BE, M), lambda i: (0, i, 0)),
        out_shape=jax.ShapeDtypeStruct((2, E, M), jnp.float32),
    )(ft, ef, *weights)


_BN = 1000


def _update_body(ns, p, wa, wb, bn, out):
    x = ns[...]
    agg = p[0] + p[1]
    out[...] = (x + bn[...]
                + jnp.dot(x, wa[...], preferred_element_type=jnp.float32)
                + jnp.dot(agg, wb[...], preferred_element_type=jnp.float32))


def _tc_update(ns, parts, Wn, bn):
    grid = (N // _BN,)
    full = lambda a: pl.BlockSpec(a.shape, lambda i: (0,) * a.ndim)
    wa, wb, bnr = Wn[:D], Wn[D:], bn.reshape(1, D)
    return pl.pallas_call(
        _update_body,
        grid=grid,
        in_specs=[pl.BlockSpec((_BN, D), lambda i: (i, 0)),
                  pl.BlockSpec((2, _BN, M), lambda i: (0, i, 0)),
                  full(wa), full(wb), full(bnr)],
        out_specs=pl.BlockSpec((_BN, D), lambda i: (i, 0)),
        out_shape=jax.ShapeDtypeStruct((N, D), jnp.float32),
    )(ns, parts, wa, wb, bnr)


def kernel(node_states, from_idx, to_idx, edge_features,
           Wm1, bm1, Wm2, bm2, Wr1, br1, Wr2, br2, Wn, bn):
    gidx = jnp.concatenate([from_idx, to_idx])
    sidx = jnp.concatenate([to_idx, from_idx])
    ft = _sc_gather(node_states, gidx.reshape(NW, _NG, _CH)).reshape(2, E, D)
    msg = _tc_messages(ft, edge_features.astype(jnp.bfloat16),
                       Wm1, bm1, Wm2, bm2, Wr1, br1, Wr2, br2)
    sidx4 = jnp.pad(sidx.reshape(NW, _NG, _CH),
                    ((0, 0), (0, _NK * _IC - _NG), (0, 0))
                    ).reshape(NW, _NK, _IC, _CH)
    parts = _sc_scatter(msg.reshape(2 * E, M), sidx4,
                        jnp.zeros((_NPAD, M), jnp.float32))
    parts = parts.reshape(NC, _NPAD, M)
    return _tc_update(node_states, parts, Wn, bn)


# speedup vs baseline: 3.3752x; 1.2504x over previous
import functools

import jax
import jax.numpy as jnp
from jax import lax
from jax.experimental import pallas as pl
from jax.experimental.pallas import tpu as pltpu
from jax.experimental.pallas import tpu_sc as plsc

N = 10000
E = 320000
D = 128
DE = 16
H = 256
M = 128

NC = 2
NS = 16
NW = NC * NS

_K = 5
_EK = E // _K
_RW = (2 * _EK) // NW
_CH = 80
_NG = _RW // _CH
_NP = _NG // 2
_IC = 8
_NK = 7

_NPAD = 10240
_N_PER_S = _NPAD // NS


def _gather_body(table_hbm, idx_hbm, out_hbm, idx_res, rows0, rows1, s0, s1):
    c = lax.axis_index("c")
    s = lax.axis_index("s")
    wid = s * NC + c
    base0 = wid * _RW

    pltpu.sync_copy(idx_hbm.at[wid], idx_res)

    pltpu.async_copy(table_hbm.at[idx_res.at[0]], rows0, s0)

    def step(p, _):
        g = 2 * p
        pltpu.make_async_copy(table_hbm.at[idx_res.at[g]], rows0, s0).wait()
        pltpu.async_copy(table_hbm.at[idx_res.at[g + 1]], rows1, s1)
        pltpu.sync_copy(rows0, out_hbm.at[pl.ds(base0 + g * _CH, _CH)])
        pltpu.make_async_copy(table_hbm.at[idx_res.at[g + 1]], rows1, s1).wait()

        @pl.when(g + 2 < _NG)
        def _():
            pltpu.async_copy(table_hbm.at[idx_res.at[g + 2]], rows0, s0)

        pltpu.sync_copy(rows1, out_hbm.at[pl.ds(base0 + (g + 1) * _CH, _CH)])
        return 0

    lax.fori_loop(0, _NP, step, 0)


def _sc_gather(table, idx):
    mesh = plsc.VectorSubcoreMesh(core_axis_name="c", subcore_axis_name="s")
    f = functools.partial(
        pl.kernel,
        mesh=mesh,
        out_type=jax.ShapeDtypeStruct((2 * _EK, D), jnp.float32),
        scratch_types=[
            pltpu.VMEM((_NG, _CH), jnp.int32),
            pltpu.VMEM((_CH, D), jnp.float32),
            pltpu.VMEM((_CH, D), jnp.float32),
            pltpu.SemaphoreType.DMA,
            pltpu.SemaphoreType.DMA,
        ],
    )(_gather_body)
    return f(table, idx)


def _scatter_body(msg_hbm, sidx_hbm, z_hbm, out_hbm,
                  idx_v, msg0, msg1, s0, s1, acc_sh):
    c = lax.axis_index("c")
    s = lax.axis_index("s")
    wid = s * NC + c
    base0 = wid * _RW

    @pl.when(s == 0)
    def _():
        pltpu.sync_copy(z_hbm, acc_sh)

    plsc.subcore_barrier()

    pltpu.async_copy(msg_hbm.at[pl.ds(base0, _CH)], msg0, s0)
    msgs = (msg0, msg1)
    sems = (s0, s1)

    def step(k, _):
        pltpu.sync_copy(sidx_hbm.at[wid, k], idx_v)
        for j in range(_IC):
            g = k * _IC + j
            b = j % 2
            pltpu.make_async_copy(
                msg_hbm.at[pl.ds(base0 + g * _CH, _CH)], msgs[b], sems[b]).wait()
            pltpu.async_copy(
                msg_hbm.at[pl.ds(base0 + (g + 1) * _CH, _CH)],
                msgs[1 - b], sems[1 - b])
            pltpu.sync_copy(msgs[b], acc_sh.at[idx_v.at[j]], add=True)
        return 0

    lax.fori_loop(0, _NK - 1, step, 0)

    pltpu.sync_copy(sidx_hbm.at[wid, _NK - 1], idx_v)
    g0 = (_NK - 1) * _IC
    pltpu.make_async_copy(
        msg_hbm.at[pl.ds(base0 + g0 * _CH, _CH)], msg0, s0).wait()
    pltpu.async_copy(
        msg_hbm.at[pl.ds(base0 + (g0 + 1) * _CH, _CH)], msg1, s1)
    pltpu.sync_copy(msg0, acc_sh.at[idx_v.at[0]], add=True)
    pltpu.make_async_copy(
        msg_hbm.at[pl.ds(base0 + (g0 + 1) * _CH, _CH)], msg1, s1).wait()
    pltpu.sync_copy(msg1, acc_sh.at[idx_v.at[1]], add=True)

    plsc.subcore_barrier()
    pltpu.sync_copy(
        acc_sh.at[pl.ds(s * _N_PER_S, _N_PER_S)],
        out_hbm.at[pl.ds(c * _NPAD + s * _N_PER_S, _N_PER_S)],
    )


def _sc_scatter(msg, sidx, zeros_nm):
    mesh = plsc.VectorSubcoreMesh(core_axis_name="c", subcore_axis_name="s")
    f = functools.partial(
        pl.kernel,
        mesh=mesh,
        out_type=jax.ShapeDtypeStruct((NC * _NPAD, M), jnp.float32),
        scratch_types=[
            pltpu.VMEM((_IC, _CH), jnp.int32),
            pltpu.VMEM((_CH, M), jnp.float32),
            pltpu.VMEM((_CH, M), jnp.float32),
            pltpu.SemaphoreType.DMA,
            pltpu.SemaphoreType.DMA,
            pltpu.VMEM_SHARED((_NPAD, M), jnp.float32),
        ],
    )(_scatter_body)
    return f(msg, sidx, zeros_nm)


_BE = 4000


def _msg_body(ft, ef, wm1f, wm1t, wm1e, bm1, wm2, bm2,
              wr1f, wr1t, wr1e, br1, wr2, br2, out):
    f = ft[0].astype(jnp.bfloat16)
    t = ft[1].astype(jnp.bfloat16)
    e = ef[...]
    hf = jnp.maximum(
        jnp.dot(f, wm1f[...], preferred_element_type=jnp.float32)
        + jnp.dot(t, wm1t[...], preferred_element_type=jnp.float32)
        + jnp.dot(e, wm1e[...], preferred_element_type=jnp.float32)
        + bm1[...], 0.0).astype(jnp.bfloat16)
    out[0] = jnp.dot(hf, wm2[...], preferred_element_type=jnp.float32) + bm2[...]
    hr = jnp.maximum(
        jnp.dot(t, wr1f[...], preferred_element_type=jnp.float32)
        + jnp.dot(f, wr1t[...], preferred_element_type=jnp.float32)
        + jnp.dot(e, wr1e[...], preferred_element_type=jnp.float32)
        + br1[...], 0.0).astype(jnp.bfloat16)
    out[1] = jnp.dot(hr, wr2[...], preferred_element_type=jnp.float32) + br2[...]


def _tc_messages(ft, ef, weights):
    grid = (_EK // _BE,)
    full = lambda a: pl.BlockSpec(a.shape, lambda i: (0,) * a.ndim)
    return pl.pallas_call(
        _msg_body,
        grid=grid,
        in_specs=[pl.BlockSpec((2, _BE, D), lambda i: (0, i, 0)),
                  pl.BlockSpec((_BE, DE), lambda i: (i, 0))]
                 + [full(w) for w in weights],
        out_specs=pl.BlockSpec((2, _BE, M), lambda i: (0, i, 0)),
        out_shape=jax.ShapeDtypeStruct((2, _EK, M), jnp.float32),
    )(ft, ef, *weights)


_BN = 1000


def _update_body(ns, p, wa, wb, bn, out):
    x = ns[...]
    agg = jnp.sum(p[...], axis=(0, 1))
    out[...] = (x + bn[...]
                + jnp.dot(x, wa[...], preferred_element_type=jnp.float32)
                + jnp.dot(agg, wb[...], preferred_element_type=jnp.float32))


def _tc_update(ns, parts, Wn, bn):
    grid = (N // _BN,)
    full = lambda a: pl.BlockSpec(a.shape, lambda i: (0,) * a.ndim)
    wa, wb, bnr = Wn[:D], Wn[D:], bn.reshape(1, D)
    return pl.pallas_call(
        _update_body,
        grid=grid,
        in_specs=[pl.BlockSpec((_BN, D), lambda i: (i, 0)),
                  pl.BlockSpec((_K, NC, _BN, M), lambda i: (0, 0, i, 0)),
                  full(wa), full(wb), full(bnr)],
        out_specs=pl.BlockSpec((_BN, D), lambda i: (i, 0)),
        out_shape=jax.ShapeDtypeStruct((N, D), jnp.float32),
    )(ns, parts, wa, wb, bnr)


def kernel(node_states, from_idx, to_idx, edge_features,
           Wm1, bm1, Wm2, bm2, Wr1, br1, Wr2, br2, Wn, bn):
    b16 = lambda a: a.astype(jnp.bfloat16)
    weights = [b16(Wm1[:D]), b16(Wm1[D:2 * D]), b16(Wm1[2 * D:]),
               bm1.reshape(1, H), b16(Wm2), bm2.reshape(1, M),
               b16(Wr1[:D]), b16(Wr1[D:2 * D]), b16(Wr1[2 * D:]),
               br1.reshape(1, H), b16(Wr2), br2.reshape(1, M)]
    zeros_nm = jnp.zeros((_NPAD, M), jnp.float32)
    ef16 = edge_features.astype(jnp.bfloat16)

    parts = []
    for k in range(_K):
        fr = lax.dynamic_slice_in_dim(from_idx, k * _EK, _EK)
        to = lax.dynamic_slice_in_dim(to_idx, k * _EK, _EK)
        gidx = jnp.concatenate([fr, to]).reshape(NW, _NG, _CH)
        sidx = jnp.pad(
            jnp.concatenate([to, fr]).reshape(NW, _NG, _CH),
            ((0, 0), (0, _NK * _IC - _NG), (0, 0)),
        ).reshape(NW, _NK, _IC, _CH)
        ft = _sc_gather(node_states, gidx).reshape(2, _EK, D)
        efk = lax.dynamic_slice_in_dim(ef16, k * _EK, _EK)
        msg = _tc_messages(ft, efk, weights)
        parts.append(_sc_scatter(msg.reshape(2 * _EK, M), sidx, zeros_nm))

    parts = jnp.stack(parts).reshape(_K, NC, _NPAD, M)
    return _tc_update(node_states, parts, Wn, bn)


# speedup vs baseline: 3.5403x; 1.0489x over previous
import functools

import jax
import jax.numpy as jnp
from jax import lax
from jax.experimental import pallas as pl
from jax.experimental.pallas import tpu as pltpu
from jax.experimental.pallas import tpu_sc as plsc

N = 10000
E = 320000
D = 128
DE = 16
H = 256
M = 128

NC = 2
NS = 16
NW = NC * NS

_K = 5
_EK = E // _K
_RW = (2 * _EK) // NW
_CH = 80
_NG = _RW // _CH

_NPAD = 10240
_N_PER_S = _NPAD // NS


_GB = 4


def _gather_body(table_hbm, idx_hbm, out_hbm,
                 idx_res, r0, r1, r2, r3,
                 g0, g1, g2, g3, t0, t1, t2, t3):
    c = lax.axis_index("c")
    s = lax.axis_index("s")
    wid = s * NC + c
    base0 = wid * _RW
    rows = (r0, r1, r2, r3)
    gsem = (g0, g1, g2, g3)
    tsem = (t0, t1, t2, t3)

    pltpu.sync_copy(idx_hbm.at[wid], idx_res)

    def fetch(g, b):
        pltpu.async_copy(table_hbm.at[idx_res.at[g]], rows[b], gsem[b])

    def fetch_wait(g, b):
        pltpu.make_async_copy(
            table_hbm.at[idx_res.at[g]], rows[b], gsem[b]).wait()

    def store(g, b):
        pltpu.async_copy(
            rows[b], out_hbm.at[pl.ds(base0 + g * _CH, _CH)], tsem[b])

    def store_wait(g, b):
        pltpu.make_async_copy(
            rows[b], out_hbm.at[pl.ds(base0 + g * _CH, _CH)], tsem[b]).wait()

    def step(q, _):
        for j in range(_GB):
            g = q * _GB + j

            @pl.when(g >= _GB)
            def _():
                store_wait(g - _GB, j)

            fetch(g, j)
            b2 = (j - 2) % _GB

            @pl.when(g >= 2)
            def _():
                fetch_wait(g - 2, b2)
                store(g - 2, b2)
        return 0

    nq = _NG // _GB
    lax.fori_loop(0, nq, step, 0)

    for g in (nq * _GB, nq * _GB + 1):
        b = g % _GB
        store_wait(g - _GB, b)
        fetch(g, b)
    for g in range(_NG - _GB, _NG):
        fetch_wait(g, g % _GB)
        store(g, g % _GB)
    for g in range(_NG - _GB, _NG):
        store_wait(g, g % _GB)


def _sc_gather(table, idx):
    mesh = plsc.VectorSubcoreMesh(core_axis_name="c", subcore_axis_name="s")
    f = functools.partial(
        pl.kernel,
        mesh=mesh,
        out_type=jax.ShapeDtypeStruct((2 * _EK, D), jnp.float32),
        scratch_types=[pltpu.VMEM((_NG, _CH), jnp.int32)]
                      + [pltpu.VMEM((_CH, D), jnp.float32)] * _GB
                      + [pltpu.SemaphoreType.DMA] * (2 * _GB),
    )(_gather_body)
    return f(table, idx)


_SB = 3


def _scatter_body(msg_hbm, sidx_hbm, z_hbm, out_hbm,
                  idx_res, m0, m1, m2, l0, l1, l2, a0, a1, a2, acc_sh):
    c = lax.axis_index("c")
    s = lax.axis_index("s")
    wid = s * NC + c
    base0 = wid * _RW
    msgs = (m0, m1, m2)
    lsem = (l0, l1, l2)
    asem = (a0, a1, a2)

    @pl.when(s == 0)
    def _():
        pltpu.sync_copy(z_hbm, acc_sh)

    pltpu.sync_copy(sidx_hbm.at[wid], idx_res)
    plsc.subcore_barrier()

    def load(g, b):
        pltpu.async_copy(
            msg_hbm.at[pl.ds(base0 + g * _CH, _CH)], msgs[b], lsem[b])

    def load_wait(g, b):
        pltpu.make_async_copy(
            msg_hbm.at[pl.ds(base0 + g * _CH, _CH)], msgs[b], lsem[b]).wait()

    def scat(g, b):
        pltpu.async_copy(msgs[b], acc_sh.at[idx_res.at[g]], asem[b], add=True)

    def scat_wait(g, b):
        pltpu.make_async_copy(msgs[b], acc_sh.at[idx_res.at[g]], asem[b]).wait()

    def step(q, _):
        for j in range(_SB):
            g = q * _SB + j

            @pl.when(g >= _SB)
            def _():
                scat_wait(g - _SB, j)

            load(g, j)
            b2 = (j - 1) % _SB

            @pl.when(g >= 1)
            def _():
                load_wait(g - 1, b2)
                scat(g - 1, b2)
        return 0

    nq = _NG // _SB
    lax.fori_loop(0, nq, step, 0)

    for g in (nq * _SB, nq * _SB + 1):
        b = g % _SB
        scat_wait(g - _SB, b)
        load(g, b)
        load_wait(g - 1, (g - 1) % _SB)
        scat(g - 1, (g - 1) % _SB)
    load_wait(_NG - 1, (_NG - 1) % _SB)
    scat(_NG - 1, (_NG - 1) % _SB)
    for g in range(_NG - _SB, _NG):
        scat_wait(g, g % _SB)

    plsc.subcore_barrier()
    pltpu.sync_copy(
        acc_sh.at[pl.ds(s * _N_PER_S, _N_PER_S)],
        out_hbm.at[pl.ds(c * _NPAD + s * _N_PER_S, _N_PER_S)],
    )


def _sc_scatter(msg, sidx, zeros_nm):
    mesh = plsc.VectorSubcoreMesh(core_axis_name="c", subcore_axis_name="s")
    f = functools.partial(
        pl.kernel,
        mesh=mesh,
        out_type=jax.ShapeDtypeStruct((NC * _NPAD, M), jnp.float32),
        scratch_types=[pltpu.VMEM((_NG, _CH), jnp.int32)]
                      + [pltpu.VMEM((_CH, M), jnp.float32)] * _SB
                      + [pltpu.SemaphoreType.DMA] * (2 * _SB)
                      + [pltpu.VMEM_SHARED((_NPAD, M), jnp.float32)],
    )(_scatter_body)
    return f(msg, sidx, zeros_nm)


_BE = 4000


def _msg_body(ft, ef, wm1f, wm1t, wm1e, bm1, wm2, bm2,
              wr1f, wr1t, wr1e, br1, wr2, br2, out):
    f = ft[0].astype(jnp.bfloat16)
    t = ft[1].astype(jnp.bfloat16)
    e = ef[...]
    hf = jnp.maximum(
        jnp.dot(f, wm1f[...], preferred_element_type=jnp.float32)
        + jnp.dot(t, wm1t[...], preferred_element_type=jnp.float32)
        + jnp.dot(e, wm1e[...], preferred_element_type=jnp.float32)
        + bm1[...], 0.0).astype(jnp.bfloat16)
    out[0] = jnp.dot(hf, wm2[...], preferred_element_type=jnp.float32) + bm2[...]
    hr = jnp.maximum(
        jnp.dot(t, wr1f[...], preferred_element_type=jnp.float32)
        + jnp.dot(f, wr1t[...], preferred_element_type=jnp.float32)
        + jnp.dot(e, wr1e[...], preferred_element_type=jnp.float32)
        + br1[...], 0.0).astype(jnp.bfloat16)
    out[1] = jnp.dot(hr, wr2[...], preferred_element_type=jnp.float32) + br2[...]


def _tc_messages(ft, ef, weights):
    grid = (_EK // _BE,)
    full = lambda a: pl.BlockSpec(a.shape, lambda i: (0,) * a.ndim)
    return pl.pallas_call(
        _msg_body,
        grid=grid,
        in_specs=[pl.BlockSpec((2, _BE, D), lambda i: (0, i, 0)),
                  pl.BlockSpec((_BE, DE), lambda i: (i, 0))]
                 + [full(w) for w in weights],
        out_specs=pl.BlockSpec((2, _BE, M), lambda i: (0, i, 0)),
        out_shape=jax.ShapeDtypeStruct((2, _EK, M), jnp.float32),
    )(ft, ef, *weights)


_BN = 1000


def _update_body(ns, p, wa, wb, bn, out):
    x = ns[...]
    agg = jnp.sum(p[...], axis=(0, 1))
    out[...] = (x + bn[...]
                + jnp.dot(x, wa[...], preferred_element_type=jnp.float32)
                + jnp.dot(agg, wb[...], preferred_element_type=jnp.float32))


def _tc_update(ns, parts, Wn, bn):
    grid = (N // _BN,)
    full = lambda a: pl.BlockSpec(a.shape, lambda i: (0,) * a.ndim)
    wa, wb, bnr = Wn[:D], Wn[D:], bn.reshape(1, D)
    return pl.pallas_call(
        _update_body,
        grid=grid,
        in_specs=[pl.BlockSpec((_BN, D), lambda i: (i, 0)),
                  pl.BlockSpec((_K, NC, _BN, M), lambda i: (0, 0, i, 0)),
                  full(wa), full(wb), full(bnr)],
        out_specs=pl.BlockSpec((_BN, D), lambda i: (i, 0)),
        out_shape=jax.ShapeDtypeStruct((N, D), jnp.float32),
    )(ns, parts, wa, wb, bnr)


def kernel(node_states, from_idx, to_idx, edge_features,
           Wm1, bm1, Wm2, bm2, Wr1, br1, Wr2, br2, Wn, bn):
    b16 = lambda a: a.astype(jnp.bfloat16)
    weights = [b16(Wm1[:D]), b16(Wm1[D:2 * D]), b16(Wm1[2 * D:]),
               bm1.reshape(1, H), b16(Wm2), bm2.reshape(1, M),
               b16(Wr1[:D]), b16(Wr1[D:2 * D]), b16(Wr1[2 * D:]),
               br1.reshape(1, H), b16(Wr2), br2.reshape(1, M)]
    zeros_nm = jnp.zeros((_NPAD, M), jnp.float32)
    ef16 = edge_features.astype(jnp.bfloat16)

    parts = []
    for k in range(_K):
        fr = lax.dynamic_slice_in_dim(from_idx, k * _EK, _EK)
        to = lax.dynamic_slice_in_dim(to_idx, k * _EK, _EK)
        gidx = jnp.concatenate([fr, to]).reshape(NW, _NG, _CH)
        sidx = jnp.concatenate([to, fr]).reshape(NW, _NG, _CH)
        ft = _sc_gather(node_states, gidx).reshape(2, _EK, D)
        efk = lax.dynamic_slice_in_dim(ef16, k * _EK, _EK)
        msg = _tc_messages(ft, efk, weights)
        parts.append(_sc_scatter(msg.reshape(2 * _EK, M), sidx, zeros_nm))

    parts = jnp.stack(parts).reshape(_K, NC, _NPAD, M)
    return _tc_update(node_states, parts, Wn, bn)


# speedup vs baseline: 4.3540x; 1.2299x over previous
import functools

import jax
import jax.numpy as jnp
from jax import lax
from jax.experimental import pallas as pl
from jax.experimental.pallas import tpu as pltpu
from jax.experimental.pallas import tpu_sc as plsc

N = 10000
E = 320000
D = 128
DE = 16
H = 256
M = 128

NC = 2
NS = 16
NW = NC * NS

_K = 5
_EK = E // _K
_RW = (2 * _EK) // NW
_CH = 80
_NG = _RW // _CH

_NPAD = 10240
_N_PER_S = _NPAD // NS


_GB = 4


def _gather_body(table_hbm, idx_hbm, out_hbm,
                 idx_res, r0, r1, r2, r3,
                 g0, g1, g2, g3, t0, t1, t2, t3):
    c = lax.axis_index("c")
    s = lax.axis_index("s")
    wid = s * NC + c
    base0 = wid * _RW
    rows = (r0, r1, r2, r3)
    gsem = (g0, g1, g2, g3)
    tsem = (t0, t1, t2, t3)

    pltpu.sync_copy(idx_hbm.at[wid], idx_res)

    def fetch(g, b):
        pltpu.async_copy(table_hbm.at[idx_res.at[g]], rows[b], gsem[b])

    def fetch_wait(g, b):
        pltpu.make_async_copy(
            table_hbm.at[idx_res.at[g]], rows[b], gsem[b]).wait()

    def store(g, b):
        pltpu.async_copy(
            rows[b], out_hbm.at[pl.ds(base0 + g * _CH, _CH)], tsem[b])

    def store_wait(g, b):
        pltpu.make_async_copy(
            rows[b], out_hbm.at[pl.ds(base0 + g * _CH, _CH)], tsem[b]).wait()

    def step(q, _):
        for j in range(_GB):
            g = q * _GB + j

            @pl.when(g >= _GB)
            def _():
                store_wait(g - _GB, j)

            fetch(g, j)
            b2 = (j - 2) % _GB

            @pl.when(g >= 2)
            def _():
                fetch_wait(g - 2, b2)
                store(g - 2, b2)
        return 0

    nq = _NG // _GB
    lax.fori_loop(0, nq, step, 0)

    for g in (nq * _GB, nq * _GB + 1):
        b = g % _GB
        store_wait(g - _GB, b)
        fetch(g, b)
    for g in range(_NG - _GB, _NG):
        fetch_wait(g, g % _GB)
        store(g, g % _GB)
    for g in range(_NG - _GB, _NG):
        store_wait(g, g % _GB)


def _sc_gather(table, idx):
    mesh = plsc.VectorSubcoreMesh(core_axis_name="c", subcore_axis_name="s")
    f = functools.partial(
        pl.kernel,
        mesh=mesh,
        out_type=jax.ShapeDtypeStruct((2 * _EK, D), jnp.float32),
        scratch_types=[pltpu.VMEM((_NG, _CH), jnp.int32)]
                      + [pltpu.VMEM((_CH, D), jnp.float32)] * _GB
                      + [pltpu.SemaphoreType.DMA] * (2 * _GB),
    )(_gather_body)
    return f(table, idx)


_SB = 3


def _scatter_body(msg_hbm, sidx_hbm, z_hbm, out_hbm,
                  idx_res, m0, m1, m2, l0, l1, l2, a0, a1, a2, acc_sh):
    c = lax.axis_index("c")
    s = lax.axis_index("s")
    wid = s * NC + c
    base0 = wid * _RW
    msgs = (m0, m1, m2)
    lsem = (l0, l1, l2)
    asem = (a0, a1, a2)

    @pl.when(s == 0)
    def _():
        pltpu.sync_copy(z_hbm, acc_sh)

    pltpu.sync_copy(sidx_hbm.at[wid], idx_res)
    plsc.subcore_barrier()

    def load(g, b):
        pltpu.async_copy(
            msg_hbm.at[pl.ds(base0 + g * _CH, _CH)], msgs[b], lsem[b])

    def load_wait(g, b):
        pltpu.make_async_copy(
            msg_hbm.at[pl.ds(base0 + g * _CH, _CH)], msgs[b], lsem[b]).wait()

    def scat(g, b):
        pltpu.async_copy(msgs[b], acc_sh.at[idx_res.at[g]], asem[b], add=True)

    def scat_wait(g, b):
        pltpu.make_async_copy(msgs[b], acc_sh.at[idx_res.at[g]], asem[b]).wait()

    def step(q, _):
        for j in range(_SB):
            g = q * _SB + j

            @pl.when(g >= _SB)
            def _():
                scat_wait(g - _SB, j)

            load(g, j)
            b2 = (j - 1) % _SB

            @pl.when(g >= 1)
            def _():
                load_wait(g - 1, b2)
                scat(g - 1, b2)
        return 0

    nq = _NG // _SB
    lax.fori_loop(0, nq, step, 0)

    for g in (nq * _SB, nq * _SB + 1):
        b = g % _SB
        scat_wait(g - _SB, b)
        load(g, b)
        load_wait(g - 1, (g - 1) % _SB)
        scat(g - 1, (g - 1) % _SB)
    load_wait(_NG - 1, (_NG - 1) % _SB)
    scat(_NG - 1, (_NG - 1) % _SB)
    for g in range(_NG - _SB, _NG):
        scat_wait(g, g % _SB)

    plsc.subcore_barrier()
    pltpu.sync_copy(
        acc_sh.at[pl.ds(s * _N_PER_S, _N_PER_S)],
        out_hbm.at[pl.ds(c * _NPAD + s * _N_PER_S, _N_PER_S)],
    )


def _sc_scatter(msg, sidx, zeros_nm):
    mesh = plsc.VectorSubcoreMesh(core_axis_name="c", subcore_axis_name="s")
    f = functools.partial(
        pl.kernel,
        mesh=mesh,
        out_type=jax.ShapeDtypeStruct((NC * _NPAD, M), jnp.float32),
        scratch_types=[pltpu.VMEM((_NG, _CH), jnp.int32)]
                      + [pltpu.VMEM((_CH, M), jnp.float32)] * _SB
                      + [pltpu.SemaphoreType.DMA] * (2 * _SB)
                      + [pltpu.VMEM_SHARED((_NPAD, M), jnp.float32)],
    )(_scatter_body)
    return f(msg, sidx, zeros_nm)


_BE = 6400


_TDN = (((0,), (0,)), ((), ()))


def _msg_body(ft, eft, wm1f, wm1t, wm1e, bm1, wm2, bm2,
              wr1f, wr1t, wr1e, br1, wr2, br2, out):
    f = ft[0].astype(jnp.bfloat16)
    t = ft[1].astype(jnp.bfloat16)
    e = eft[...].astype(jnp.bfloat16)
    hf = jnp.maximum(
        jnp.dot(f, wm1f[...], preferred_element_type=jnp.float32)
        + jnp.dot(t, wm1t[...], preferred_element_type=jnp.float32)
        + lax.dot_general(e, wm1e[...], _TDN,
                          preferred_element_type=jnp.float32)
        + bm1[...], 0.0).astype(jnp.bfloat16)
    out[0] = jnp.dot(hf, wm2[...], preferred_element_type=jnp.float32) + bm2[...]
    hr = jnp.maximum(
        jnp.dot(t, wr1f[...], preferred_element_type=jnp.float32)
        + jnp.dot(f, wr1t[...], preferred_element_type=jnp.float32)
        + lax.dot_general(e, wr1e[...], _TDN,
                          preferred_element_type=jnp.float32)
        + br1[...], 0.0).astype(jnp.bfloat16)
    out[1] = jnp.dot(hr, wr2[...], preferred_element_type=jnp.float32) + br2[...]


def _tc_messages(ft, eft, weights, k):
    grid = (_EK // _BE,)
    full = lambda a: pl.BlockSpec(a.shape, lambda i: (0,) * a.ndim)
    off = k * (_EK // _BE)
    return pl.pallas_call(
        _msg_body,
        grid=grid,
        in_specs=[pl.BlockSpec((2, _BE, D), lambda i: (0, i, 0)),
                  pl.BlockSpec((DE, _BE), lambda i: (0, i + off))]
                 + [full(w) for w in weights],
        out_specs=pl.BlockSpec((2, _BE, M), lambda i: (0, i, 0)),
        out_shape=jax.ShapeDtypeStruct((2, _EK, M), jnp.float32),
    )(ft, eft, *weights)


_BN = 1000


def _update_body(ns, p, wa, wb, bn, out):
    x = ns[...]
    agg = jnp.sum(p[...], axis=(0, 1))
    out[...] = (x + bn[...]
                + jnp.dot(x, wa[...], preferred_element_type=jnp.float32)
                + jnp.dot(agg, wb[...], preferred_element_type=jnp.float32))


def _tc_update(ns, parts, Wn, bn):
    grid = (N // _BN,)
    full = lambda a: pl.BlockSpec(a.shape, lambda i: (0,) * a.ndim)
    wa, wb, bnr = Wn[:D], Wn[D:], bn.reshape(1, D)
    return pl.pallas_call(
        _update_body,
        grid=grid,
        in_specs=[pl.BlockSpec((_BN, D), lambda i: (i, 0)),
                  pl.BlockSpec((_K, NC, _BN, M), lambda i: (0, 0, i, 0)),
                  full(wa), full(wb), full(bnr)],
        out_specs=pl.BlockSpec((_BN, D), lambda i: (i, 0)),
        out_shape=jax.ShapeDtypeStruct((N, D), jnp.float32),
    )(ns, parts, wa, wb, bnr)


def kernel(node_states, from_idx, to_idx, edge_features,
           Wm1, bm1, Wm2, bm2, Wr1, br1, Wr2, br2, Wn, bn):
    b16 = lambda a: a.astype(jnp.bfloat16)
    weights = [b16(Wm1[:D]), b16(Wm1[D:2 * D]), b16(Wm1[2 * D:]),
               bm1.reshape(1, H), b16(Wm2), bm2.reshape(1, M),
               b16(Wr1[:D]), b16(Wr1[D:2 * D]), b16(Wr1[2 * D:]),
               br1.reshape(1, H), b16(Wr2), br2.reshape(1, M)]
    zeros_nm = jnp.zeros((_NPAD, M), jnp.float32)
    eft = edge_features.T

    parts = []
    for k in range(_K):
        fr = lax.dynamic_slice_in_dim(from_idx, k * _EK, _EK)
        to = lax.dynamic_slice_in_dim(to_idx, k * _EK, _EK)
        gidx = jnp.concatenate([fr, to]).reshape(NW, _NG, _CH)
        sidx = jnp.concatenate([to, fr]).reshape(NW, _NG, _CH)
        ft = _sc_gather(node_states, gidx).reshape(2, _EK, D)
        msg = _tc_messages(ft, eft, weights, k)
        parts.append(_sc_scatter(msg.reshape(2 * _EK, M), sidx, zeros_nm))

    parts = jnp.stack(parts).reshape(_K, NC, _NPAD, M)
    return _tc_update(node_states, parts, Wn, bn)


# speedup vs baseline: 4.7781x; 1.0974x over previous
import functools

import jax
import jax.numpy as jnp
from jax import lax
from jax.experimental import pallas as pl
from jax.experimental.pallas import tpu as pltpu
from jax.experimental.pallas import tpu_sc as plsc

N = 10000
E = 320000
D = 128
DE = 16
H = 256
M = 128

NC = 2
NS = 16
NW = NC * NS

_K = 5
_EK = E // _K
_RW = (2 * _EK) // NW
_CH = 80
_NG = _RW // _CH

_NPAD = 10240
_N_PER_S = _NPAD // NS


_GB = 4


def _gather_body(table_hbm, idx_hbm, out_hbm,
                 idx_res, r0, r1, r2, r3,
                 g0, g1, g2, g3, t0, t1, t2, t3):
    c = lax.axis_index("c")
    s = lax.axis_index("s")
    wid = s * NC + c
    rows = (r0, r1, r2, r3)
    gsem = (g0, g1, g2, g3)
    tsem = (t0, t1, t2, t3)

    orow = (wid % NS) * _RW
    ocol = pl.multiple_of(jnp.where(wid < NS, 0, D), D)

    pltpu.sync_copy(idx_hbm.at[wid], idx_res)

    def fetch(g, b):
        pltpu.async_copy(table_hbm.at[idx_res.at[g]], rows[b], gsem[b])

    def fetch_wait(g, b):
        pltpu.make_async_copy(
            table_hbm.at[idx_res.at[g]], rows[b], gsem[b]).wait()

    def store(g, b):
        pltpu.async_copy(
            rows[b], out_hbm.at[pl.ds(orow + g * _CH, _CH), pl.ds(ocol, D)],
            tsem[b])

    def store_wait(g, b):
        pltpu.make_async_copy(
            rows[b], out_hbm.at[pl.ds(orow + g * _CH, _CH), pl.ds(ocol, D)],
            tsem[b]).wait()

    def step(q, _):
        for j in range(_GB):
            g = q * _GB + j

            @pl.when(g >= _GB)
            def _():
                store_wait(g - _GB, j)

            fetch(g, j)
            b2 = (j - 2) % _GB

            @pl.when(g >= 2)
            def _():
                fetch_wait(g - 2, b2)
                store(g - 2, b2)
        return 0

    nq = _NG // _GB
    lax.fori_loop(0, nq, step, 0)

    for g in (nq * _GB, nq * _GB + 1):
        b = g % _GB
        store_wait(g - _GB, b)
        fetch(g, b)
    for g in range(_NG - _GB, _NG):
        fetch_wait(g, g % _GB)
        store(g, g % _GB)
    for g in range(_NG - _GB, _NG):
        store_wait(g, g % _GB)


def _sc_gather(table, idx):
    mesh = plsc.VectorSubcoreMesh(core_axis_name="c", subcore_axis_name="s")
    f = functools.partial(
        pl.kernel,
        mesh=mesh,
        out_type=jax.ShapeDtypeStruct((_EK, 2 * D), jnp.float32),
        scratch_types=[pltpu.VMEM((_NG, _CH), jnp.int32)]
                      + [pltpu.VMEM((_CH, D), jnp.float32)] * _GB
                      + [pltpu.SemaphoreType.DMA] * (2 * _GB),
    )(_gather_body)
    return f(table, idx)


_SB = 3


def _scatter_body(msg_hbm, sidx_hbm, z_hbm, out_hbm,
                  idx_res, m0, m1, m2, l0, l1, l2, a0, a1, a2, acc_sh):
    c = lax.axis_index("c")
    s = lax.axis_index("s")
    wid = s * NC + c
    base0 = wid * _RW
    msgs = (m0, m1, m2)
    lsem = (l0, l1, l2)
    asem = (a0, a1, a2)

    @pl.when(s == 0)
    def _():
        pltpu.sync_copy(z_hbm.at[pl.ds(c * _NPAD, _NPAD)], acc_sh)

    pltpu.sync_copy(sidx_hbm.at[wid], idx_res)
    plsc.subcore_barrier()

    def load(g, b):
        pltpu.async_copy(
            msg_hbm.at[pl.ds(base0 + g * _CH, _CH)], msgs[b], lsem[b])

    def load_wait(g, b):
        pltpu.make_async_copy(
            msg_hbm.at[pl.ds(base0 + g * _CH, _CH)], msgs[b], lsem[b]).wait()

    def scat(g, b):
        pltpu.async_copy(msgs[b], acc_sh.at[idx_res.at[g]], asem[b], add=True)

    def scat_wait(g, b):
        pltpu.make_async_copy(msgs[b], acc_sh.at[idx_res.at[g]], asem[b]).wait()

    def step(q, _):
        for j in range(_SB):
            g = q * _SB + j

            @pl.when(g >= _SB)
            def _():
                scat_wait(g - _SB, j)

            load(g, j)
            b2 = (j - 1) % _SB

            @pl.when(g >= 1)
            def _():
                load_wait(g - 1, b2)
                scat(g - 1, b2)
        return 0

    nq = _NG // _SB
    lax.fori_loop(0, nq, step, 0)

    for g in (nq * _SB, nq * _SB + 1):
        b = g % _SB
        scat_wait(g - _SB, b)
        load(g, b)
        load_wait(g - 1, (g - 1) % _SB)
        scat(g - 1, (g - 1) % _SB)
    load_wait(_NG - 1, (_NG - 1) % _SB)
    scat(_NG - 1, (_NG - 1) % _SB)
    for g in range(_NG - _SB, _NG):
        scat_wait(g, g % _SB)

    plsc.subcore_barrier()
    pltpu.sync_copy(
        acc_sh.at[pl.ds(s * _N_PER_S, _N_PER_S)],
        out_hbm.at[pl.ds(c * _NPAD + s * _N_PER_S, _N_PER_S)],
    )


def _sc_scatter(msg, sidx, init):
    mesh = plsc.VectorSubcoreMesh(core_axis_name="c", subcore_axis_name="s")
    f = functools.partial(
        pl.kernel,
        mesh=mesh,
        out_type=jax.ShapeDtypeStruct((NC * _NPAD, M), jnp.float32),
        scratch_types=[pltpu.VMEM((_NG, _CH), jnp.int32)]
                      + [pltpu.VMEM((_CH, M), jnp.float32)] * _SB
                      + [pltpu.SemaphoreType.DMA] * (2 * _SB)
                      + [pltpu.VMEM_SHARED((_NPAD, M), jnp.float32)],
    )(_scatter_body)
    return f(msg, sidx, init)


_BE = 6400


_TDN = (((0,), (0,)), ((), ()))


def _msg_body(ft, eft, u1m, wm1e, bm1, wm2, bm2,
              u1r, wr1e, br1, wr2, br2, out):
    x = ft[...].astype(jnp.bfloat16)
    e = eft[...].astype(jnp.bfloat16)
    hf = jnp.maximum(
        jnp.dot(x, u1m[...], preferred_element_type=jnp.float32)
        + lax.dot_general(e, wm1e[...], _TDN,
                          preferred_element_type=jnp.float32)
        + bm1[...], 0.0).astype(jnp.bfloat16)
    out[0] = jnp.dot(hf, wm2[...], preferred_element_type=jnp.float32) + bm2[...]
    hr = jnp.maximum(
        jnp.dot(x, u1r[...], preferred_element_type=jnp.float32)
        + lax.dot_general(e, wr1e[...], _TDN,
                          preferred_element_type=jnp.float32)
        + br1[...], 0.0).astype(jnp.bfloat16)
    out[1] = jnp.dot(hr, wr2[...], preferred_element_type=jnp.float32) + br2[...]


def _tc_messages(ft, eft, weights, k):
    grid = (_EK // _BE,)
    full = lambda a: pl.BlockSpec(a.shape, lambda i: (0,) * a.ndim)
    off = k * (_EK // _BE)
    return pl.pallas_call(
        _msg_body,
        grid=grid,
        in_specs=[pl.BlockSpec((_BE, 2 * D), lambda i: (i, 0)),
                  pl.BlockSpec((DE, _BE), lambda i: (0, i + off))]
                 + [full(w) for w in weights],
        out_specs=pl.BlockSpec((2, _BE, M), lambda i: (0, i, 0)),
        out_shape=jax.ShapeDtypeStruct((2, _EK, M), jnp.float32),
    )(ft, eft, *weights)


_BN = 1000


def _update_body(ns, p, wa, wb, bn, out):
    x = ns[...]
    agg = p[0] + p[1]
    out[...] = (x + bn[...]
                + jnp.dot(x, wa[...], preferred_element_type=jnp.float32)
                + jnp.dot(agg, wb[...], preferred_element_type=jnp.float32))


def _tc_update(ns, parts, Wn, bn):
    grid = (N // _BN,)
    full = lambda a: pl.BlockSpec(a.shape, lambda i: (0,) * a.ndim)
    wa, wb, bnr = Wn[:D], Wn[D:], bn.reshape(1, D)
    return pl.pallas_call(
        _update_body,
        grid=grid,
        in_specs=[pl.BlockSpec((_BN, D), lambda i: (i, 0)),
                  pl.BlockSpec((NC, _BN, M), lambda i: (0, i, 0)),
                  full(wa), full(wb), full(bnr)],
        out_specs=pl.BlockSpec((_BN, D), lambda i: (i, 0)),
        out_shape=jax.ShapeDtypeStruct((N, D), jnp.float32),
    )(ns, parts, wa, wb, bnr)


def kernel(node_states, from_idx, to_idx, edge_features,
           Wm1, bm1, Wm2, bm2, Wr1, br1, Wr2, br2, Wn, bn):
    b16 = lambda a: a.astype(jnp.bfloat16)
    u1r = jnp.concatenate([Wr1[D:2 * D], Wr1[:D]], axis=0)
    weights = [b16(Wm1[:2 * D]), b16(Wm1[2 * D:]), bm1.reshape(1, H),
               b16(Wm2), bm2.reshape(1, M),
               b16(u1r), b16(Wr1[2 * D:]), br1.reshape(1, H),
               b16(Wr2), br2.reshape(1, M)]
    part = jnp.zeros((NC * _NPAD, M), jnp.float32)
    eft = edge_features.T

    for k in range(_K):
        fr = lax.dynamic_slice_in_dim(from_idx, k * _EK, _EK)
        to = lax.dynamic_slice_in_dim(to_idx, k * _EK, _EK)
        gidx = jnp.concatenate([fr, to]).reshape(NW, _NG, _CH)
        sidx = jnp.concatenate([to, fr]).reshape(NW, _NG, _CH)
        ft = _sc_gather(node_states, gidx)
        msg = _tc_messages(ft, eft, weights, k)
        part = _sc_scatter(msg.reshape(2 * _EK, M), sidx, part)

    return _tc_update(node_states, part.reshape(NC, _NPAD, M), Wn, bn)
